# Initial kernel scaffold; baseline (speedup 1.0000x reference)
import jax
import jax.numpy as jnp
from jax.experimental import pallas as pl


def kernel(inputs, W_gate):
    raise NotImplementedError("write your pallas kernel here")


# speedup vs baseline: 4.3138x; 4.3138x over previous
import jax
import jax.numpy as jnp
from jax.experimental import pallas as pl
from jax.experimental.pallas import tpu as pltpu

E_EXPERTS = 64
K_TOP = 8
D_IN = 4096
T_TOKENS = 16384
BT = 512


def _gate_kernel(x_ref, w_ref, ew_ref, idx_ref, logits_ref, raw_ref):
    x = x_ref[...]
    w = w_ref[...]
    logits = jnp.dot(x, w, preferred_element_type=jnp.float32)
    logits_ref[...] = logits

    t, e = logits.shape
    iota = jax.lax.broadcasted_iota(jnp.int32, (t, e), 1)
    work = logits
    mask = jnp.zeros((t, e), jnp.bool_)
    idx_cols = []
    for _ in range(K_TOP):
        m = jnp.max(work, axis=1, keepdims=True)
        is_max = work == m
        arg = jnp.min(jnp.where(is_max, iota, e), axis=1, keepdims=True)
        sel = iota == arg
        mask = jnp.logical_or(mask, sel)
        work = jnp.where(sel, -jnp.inf, work)
        idx_cols.append(arg)
    idx_ref[...] = jnp.concatenate(idx_cols, axis=1)

    m1 = jnp.max(logits, axis=1, keepdims=True)
    ex = jnp.exp(logits - m1)
    raw_ref[...] = ex / jnp.sum(ex, axis=1, keepdims=True)
    ex_top = jnp.where(mask, ex, 0.0)
    ew_ref[...] = ex_top / jnp.sum(ex_top, axis=1, keepdims=True)


def kernel(inputs, W_gate):
    t, d = inputs.shape
    e = W_gate.shape[1]
    grid = (t // BT,)
    out_shapes = (
        jax.ShapeDtypeStruct((t, e), jnp.float32),
        jax.ShapeDtypeStruct((t, K_TOP), jnp.int32),
        jax.ShapeDtypeStruct((t, e), jnp.float32),
        jax.ShapeDtypeStruct((t, e), jnp.float32),
    )
    row_spec = pl.BlockSpec((BT, e), lambda i: (i, 0))
    out = pl.pallas_call(
        _gate_kernel,
        grid=grid,
        in_specs=[
            pl.BlockSpec((BT, d), lambda i: (i, 0)),
            pl.BlockSpec((d, e), lambda i: (0, 0)),
        ],
        out_specs=(
            row_spec,
            pl.BlockSpec((BT, K_TOP), lambda i: (i, 0)),
            row_spec,
            row_spec,
        ),
        out_shape=out_shapes,
        compiler_params=pltpu.CompilerParams(
            dimension_semantics=("arbitrary",),
        ),
    )(inputs, W_gate)
    return out


# speedup vs baseline: 4.6083x; 1.0683x over previous
import jax
import jax.numpy as jnp
from jax.experimental import pallas as pl
from jax.experimental.pallas import tpu as pltpu

E_EXPERTS = 64
K_TOP = 8
D_IN = 4096
T_TOKENS = 16384
BT = 512


def _gate_kernel(x_ref, w_ref, ew_ref, idx_ref, logits_ref, raw_ref):
    x = x_ref[...]
    w = w_ref[...]
    logits = jnp.dot(x, w, preferred_element_type=jnp.float32)
    logits_ref[...] = logits

    t, e = logits.shape
    iota = jax.lax.broadcasted_iota(jnp.int32, (t, e), 1)
    bits = logits.view(jnp.int32)
    okey = bits ^ (jnp.right_shift(bits, 31) & jnp.int32(0x7FFFFFFF))
    packed = (okey & jnp.int32(~0x3F)) | (jnp.int32(e - 1) - iota)
    mask = jnp.zeros((t, e), jnp.bool_)
    idx_cols = []
    for _ in range(K_TOP):
        m = jnp.max(packed, axis=1, keepdims=True)
        sel = packed == m
        mask = jnp.logical_or(mask, sel)
        packed = jnp.where(sel, jnp.int32(-0x80000000), packed)
        idx_cols.append(jnp.int32(e - 1) - (m & jnp.int32(0x3F)))
    idx_ref[...] = jnp.concatenate(idx_cols, axis=1)

    m1 = jnp.max(logits, axis=1, keepdims=True)
    ex = jnp.exp(logits - m1)
    raw_ref[...] = ex / jnp.sum(ex, axis=1, keepdims=True)
    ex_top = jnp.where(mask, ex, 0.0)
    ew_ref[...] = ex_top / jnp.sum(ex_top, axis=1, keepdims=True)


def kernel(inputs, W_gate):
    t, d = inputs.shape
    e = W_gate.shape[1]
    grid = (t // BT,)
    out_shapes = (
        jax.ShapeDtypeStruct((t, e), jnp.float32),
        jax.ShapeDtypeStruct((t, K_TOP), jnp.int32),
        jax.ShapeDtypeStruct((t, e), jnp.float32),
        jax.ShapeDtypeStruct((t, e), jnp.float32),
    )
    row_spec = pl.BlockSpec((BT, e), lambda i: (i, 0))
    out = pl.pallas_call(
        _gate_kernel,
        grid=grid,
        in_specs=[
            pl.BlockSpec((BT, d), lambda i: (i, 0)),
            pl.BlockSpec((d, e), lambda i: (0, 0)),
        ],
        out_specs=(
            row_spec,
            pl.BlockSpec((BT, K_TOP), lambda i: (i, 0)),
            row_spec,
            row_spec,
        ),
        out_shape=out_shapes,
        compiler_params=pltpu.CompilerParams(
            dimension_semantics=("arbitrary",),
        ),
    )(inputs, W_gate)
    return out


# speedup vs baseline: 5.0525x; 1.0964x over previous
import jax
import jax.numpy as jnp
from jax.experimental import pallas as pl
from jax.experimental.pallas import tpu as pltpu

E_EXPERTS = 64
K_TOP = 8
D_IN = 4096
T_TOKENS = 16384
BT = 512


def _gate_kernel(x_ref, w_ref, ew_ref, idx_ref, logits_ref, raw_ref):
    x = x_ref[...]
    w = w_ref[...]
    logits = jnp.dot(x, w, preferred_element_type=jnp.float32)
    logits_ref[...] = logits

    t, e = logits.shape
    iota = jax.lax.broadcasted_iota(jnp.int32, (t, e), 1)
    bits = logits.view(jnp.int32)
    neg = bits < 0
    fill = jnp.where(neg, iota, jnp.int32(e - 1) - iota)
    packed0 = ((bits & jnp.int32(~0x3F)) | fill).view(jnp.float32)
    packed = packed0
    idx_cols = []
    m = None
    for _ in range(K_TOP):
        m = jnp.max(packed, axis=1, keepdims=True)
        packed = jnp.where(packed == m, -jnp.inf, packed)
        mb = m.view(jnp.int32)
        mf = mb & jnp.int32(0x3F)
        idx_cols.append(jnp.where(mb < 0, mf, jnp.int32(e - 1) - mf))
    idx_ref[...] = jnp.concatenate(idx_cols, axis=1)
    mask = packed0 >= m

    m1 = jnp.max(logits, axis=1, keepdims=True)
    ex = jnp.exp(logits - m1)
    raw_ref[...] = ex / jnp.sum(ex, axis=1, keepdims=True)
    ex_top = jnp.where(mask, ex, 0.0)
    ew_ref[...] = ex_top / jnp.sum(ex_top, axis=1, keepdims=True)


def kernel(inputs, W_gate):
    t, d = inputs.shape
    e = W_gate.shape[1]
    grid = (t // BT,)
    out_shapes = (
        jax.ShapeDtypeStruct((t, e), jnp.float32),
        jax.ShapeDtypeStruct((t, K_TOP), jnp.int32),
        jax.ShapeDtypeStruct((t, e), jnp.float32),
        jax.ShapeDtypeStruct((t, e), jnp.float32),
    )
    row_spec = pl.BlockSpec((BT, e), lambda i: (i, 0))
    out = pl.pallas_call(
        _gate_kernel,
        grid=grid,
        in_specs=[
            pl.BlockSpec((BT, d), lambda i: (i, 0)),
            pl.BlockSpec((d, e), lambda i: (0, 0)),
        ],
        out_specs=(
            row_spec,
            pl.BlockSpec((BT, K_TOP), lambda i: (i, 0)),
            row_spec,
            row_spec,
        ),
        out_shape=out_shapes,
        compiler_params=pltpu.CompilerParams(
            dimension_semantics=("arbitrary",),
        ),
    )(inputs, W_gate)
    return out


# speedup vs baseline: 5.3656x; 1.0620x over previous
import jax
import jax.numpy as jnp
from jax.experimental import pallas as pl
from jax.experimental.pallas import tpu as pltpu

E_EXPERTS = 64
K_TOP = 8
D_IN = 4096
T_TOKENS = 16384
BT = 1024


def _gate_kernel(x_ref, w_ref, ew_ref, idx_ref, logits_ref, raw_ref):
    x = x_ref[...]
    w = w_ref[...]
    logits = jnp.dot(x, w, preferred_element_type=jnp.float32)
    logits_ref[...] = logits

    t, e = logits.shape
    iota = jax.lax.broadcasted_iota(jnp.int32, (t, e), 1)
    bits = logits.view(jnp.int32)
    neg = bits < 0
    fill = jnp.where(neg, iota, jnp.int32(e - 1) - iota)
    packed0 = ((bits & jnp.int32(~0x3F)) | fill).view(jnp.float32)
    packed = packed0
    idx_cols = []
    m = None
    for _ in range(K_TOP):
        m = jnp.max(packed, axis=1, keepdims=True)
        packed = jnp.where(packed == m, -jnp.inf, packed)
        mb = m.view(jnp.int32)
        mf = mb & jnp.int32(0x3F)
        idx_cols.append(jnp.where(mb < 0, mf, jnp.int32(e - 1) - mf))
    idx_ref[...] = jnp.concatenate(idx_cols, axis=1)
    mask = packed0 >= m

    m1 = jnp.max(logits, axis=1, keepdims=True)
    ex = jnp.exp(logits - m1)
    raw_ref[...] = ex / jnp.sum(ex, axis=1, keepdims=True)
    ex_top = jnp.where(mask, ex, 0.0)
    ew_ref[...] = ex_top / jnp.sum(ex_top, axis=1, keepdims=True)


def kernel(inputs, W_gate):
    t, d = inputs.shape
    e = W_gate.shape[1]
    grid = (t // BT,)
    out_shapes = (
        jax.ShapeDtypeStruct((t, e), jnp.float32),
        jax.ShapeDtypeStruct((t, K_TOP), jnp.int32),
        jax.ShapeDtypeStruct((t, e), jnp.float32),
        jax.ShapeDtypeStruct((t, e), jnp.float32),
    )
    row_spec = pl.BlockSpec((BT, e), lambda i: (i, 0))
    out = pl.pallas_call(
        _gate_kernel,
        grid=grid,
        in_specs=[
            pl.BlockSpec((BT, d), lambda i: (i, 0)),
            pl.BlockSpec((d, e), lambda i: (0, 0)),
        ],
        out_specs=(
            row_spec,
            pl.BlockSpec((BT, K_TOP), lambda i: (i, 0)),
            row_spec,
            row_spec,
        ),
        out_shape=out_shapes,
        compiler_params=pltpu.CompilerParams(
            dimension_semantics=("arbitrary",),
        ),
    )(inputs, W_gate)
    return out


# speedup vs baseline: 5.3736x; 1.0015x over previous
import jax
import jax.numpy as jnp
from jax.experimental import pallas as pl
from jax.experimental.pallas import tpu as pltpu

E_EXPERTS = 64
K_TOP = 8
D_IN = 4096
T_TOKENS = 16384
BT = 1024


def _gate_kernel(x1_ref, x2_ref, w_ref, ew_ref, idx_ref, logits_ref, raw_ref):
    w = w_ref[...]
    h = x1_ref.shape[1]
    logits = jnp.dot(x1_ref[...], w[:h], preferred_element_type=jnp.float32)
    logits = logits + jnp.dot(x2_ref[...], w[h:], preferred_element_type=jnp.float32)
    logits_ref[...] = logits

    t, e = logits.shape
    iota = jax.lax.broadcasted_iota(jnp.int32, (t, e), 1)
    bits = logits.view(jnp.int32)
    neg = bits < 0
    fill = jnp.where(neg, iota, jnp.int32(e - 1) - iota)
    packed0 = ((bits & jnp.int32(~0x3F)) | fill).view(jnp.float32)
    packed = packed0
    idx_cols = []
    m = None
    for _ in range(K_TOP):
        m = jnp.max(packed, axis=1, keepdims=True)
        packed = jnp.where(packed == m, -jnp.inf, packed)
        mb = m.view(jnp.int32)
        mf = mb & jnp.int32(0x3F)
        idx_cols.append(jnp.where(mb < 0, mf, jnp.int32(e - 1) - mf))
    idx_ref[...] = jnp.concatenate(idx_cols, axis=1)
    mask = packed0 >= m

    m1 = jnp.max(logits, axis=1, keepdims=True)
    ex = jnp.exp(logits - m1)
    raw_ref[...] = ex / jnp.sum(ex, axis=1, keepdims=True)
    ex_top = jnp.where(mask, ex, 0.0)
    ew_ref[...] = ex_top / jnp.sum(ex_top, axis=1, keepdims=True)


def kernel(inputs, W_gate):
    t, d = inputs.shape
    e = W_gate.shape[1]
    grid = (t // BT,)
    out_shapes = (
        jax.ShapeDtypeStruct((t, e), jnp.float32),
        jax.ShapeDtypeStruct((t, K_TOP), jnp.int32),
        jax.ShapeDtypeStruct((t, e), jnp.float32),
        jax.ShapeDtypeStruct((t, e), jnp.float32),
    )
    row_spec = pl.BlockSpec((BT, e), lambda i: (i, 0))
    out = pl.pallas_call(
        _gate_kernel,
        grid=grid,
        in_specs=[
            pl.BlockSpec((BT, d // 2), lambda i: (i, 0)),
            pl.BlockSpec((BT, d // 2), lambda i: (i, 1)),
            pl.BlockSpec((d, e), lambda i: (0, 0)),
        ],
        out_specs=(
            row_spec,
            pl.BlockSpec((BT, K_TOP), lambda i: (i, 0)),
            row_spec,
            row_spec,
        ),
        out_shape=out_shapes,
        compiler_params=pltpu.CompilerParams(
            dimension_semantics=("arbitrary",),
        ),
    )(inputs, inputs, W_gate)
    return out
